# baseline (device time: 66590 ns/iter reference)
import jax
import jax.numpy as jnp
from jax import lax
from jax.experimental import pallas as pl
from jax.experimental.pallas import tpu as pltpu

N_Y = 4


def kernel(x, assign, W1, W2):
    t, d = x.shape
    e_loc, _, f = W1.shape

    x_b = x.astype(jnp.bfloat16)
    w1_b = W1.astype(jnp.bfloat16)
    w2_b = W2.astype(jnp.bfloat16)
    a2d = assign.reshape(t, 1)

    def body(x_ref, a_ref, w1_ref, w2_ref, out_ref,
             xg, ag, cbuf, crec,
             xsend_sems, xrecv_sems, asend_sems, arecv_sems,
             csend_sems, crecv_sems):
        my_x = lax.axis_index("x")
        my_y = lax.axis_index("y")
        my_z = lax.axis_index("z")

        def peer_id(s):
            return (my_x, (my_y + s) % N_Y, my_z)

        bar = pltpu.get_barrier_semaphore()
        for s in range(1, N_Y):
            pl.semaphore_signal(
                bar, inc=1,
                device_id=peer_id(s),
                device_id_type=pl.DeviceIdType.MESH,
            )
        pl.semaphore_wait(bar, N_Y - 1)

        x_sends = []
        a_sends = []
        for s in range(1, N_Y):
            slot = N_Y - s
            r = pltpu.make_async_remote_copy(
                src_ref=x_ref, dst_ref=xg.at[slot],
                send_sem=xsend_sems.at[s], recv_sem=xrecv_sems.at[slot],
                device_id=peer_id(s),
                device_id_type=pl.DeviceIdType.MESH,
            )
            r.start()
            x_sends.append(r)
            r = pltpu.make_async_remote_copy(
                src_ref=a_ref, dst_ref=ag.at[slot],
                send_sem=asend_sems.at[s], recv_sem=arecv_sems.at[slot],
                device_id=peer_id(s),
                device_id_type=pl.DeviceIdType.MESH,
            )
            r.start()
            a_sends.append(r)

        own = None
        c_sends = []
        for s in range(N_Y):
            if s == 0:
                xb = x_ref[...]
                ab = a_ref[...]
            else:
                rx = pltpu.make_async_remote_copy(
                    src_ref=x_ref, dst_ref=xg.at[s],
                    send_sem=xsend_sems.at[s], recv_sem=xrecv_sems.at[s],
                    device_id=peer_id(s),
                    device_id_type=pl.DeviceIdType.MESH,
                )
                rx.wait_recv()
                ra = pltpu.make_async_remote_copy(
                    src_ref=a_ref, dst_ref=ag.at[s],
                    send_sem=asend_sems.at[s], recv_sem=arecv_sems.at[s],
                    device_id=peer_id(s),
                    device_id_type=pl.DeviceIdType.MESH,
                )
                ra.wait_recv()
                xb = xg[s]
                ab = ag[s]
            acc = jnp.zeros((t, d), jnp.float32)
            for le in range(e_loc):
                ge = my_y * e_loc + le
                xm = jnp.where(ab == ge, xb, jnp.zeros_like(xb))
                h = jnp.dot(xm, w1_ref[le],
                            preferred_element_type=jnp.float32)
                h = jnp.maximum(h, 0.0).astype(jnp.bfloat16)
                acc = acc + jnp.dot(h, w2_ref[le],
                                    preferred_element_type=jnp.float32)
            if s == 0:
                own = acc
            else:
                cbuf[s] = acc.astype(jnp.bfloat16)
                r = pltpu.make_async_remote_copy(
                    src_ref=cbuf.at[s], dst_ref=crec.at[N_Y - s],
                    send_sem=csend_sems.at[s], recv_sem=crecv_sems.at[N_Y - s],
                    device_id=peer_id(s),
                    device_id_type=pl.DeviceIdType.MESH,
                )
                r.start()
                c_sends.append(r)

        total = own
        for s in range(1, N_Y):
            rc = pltpu.make_async_remote_copy(
                src_ref=cbuf.at[s], dst_ref=crec.at[s],
                send_sem=csend_sems.at[s], recv_sem=crecv_sems.at[s],
                device_id=peer_id(s),
                device_id_type=pl.DeviceIdType.MESH,
            )
            rc.wait_recv()
            total = total + crec[s].astype(jnp.float32)
        out_ref[...] = total

        for r in x_sends + a_sends + c_sends:
            r.wait_send()

    return pl.pallas_call(
        body,
        out_shape=jax.ShapeDtypeStruct((t, d), jnp.float32),
        in_specs=[pl.BlockSpec(memory_space=pltpu.VMEM)] * 4,
        out_specs=pl.BlockSpec(memory_space=pltpu.VMEM),
        scratch_shapes=[
            pltpu.VMEM((N_Y, t, d), jnp.bfloat16),
            pltpu.VMEM((N_Y, t, 1), jnp.int32),
            pltpu.VMEM((N_Y, t, d), jnp.bfloat16),
            pltpu.VMEM((N_Y, t, d), jnp.bfloat16),
            pltpu.SemaphoreType.DMA((N_Y,)),
            pltpu.SemaphoreType.DMA((N_Y,)),
            pltpu.SemaphoreType.DMA((N_Y,)),
            pltpu.SemaphoreType.DMA((N_Y,)),
            pltpu.SemaphoreType.DMA((N_Y,)),
            pltpu.SemaphoreType.DMA((N_Y,)),
        ],
        compiler_params=pltpu.CompilerParams(collective_id=0),
    )(x_b, a2d, w1_b, w2_b)
